# baseline (speedup 1.0000x reference)
import functools

import jax
import jax.numpy as jnp
from jax import lax
from jax.experimental import pallas as pl
from jax.experimental.pallas import tpu as pltpu
from jax.experimental.pallas import tpu_sc as plsc

D_MODEL = 64
D_PAD = 128
SCALE = 8.0

_info = plsc.get_sparse_core_info()
_NC, _NS, _L = _info.num_cores, _info.num_subcores, _info.num_lanes
_NW = _NC * _NS

CHUNK = 128
BV = 4096


def _prep_body(tt_ref, out_ref):
    at = jnp.transpose(tt_ref[...]) * SCALE
    out_ref[...] = jnp.concatenate([at, at], axis=1)


def _prep_table(table):
    v = table.shape[0]
    tt = table.T
    return pl.pallas_call(
        _prep_body,
        grid=(pl.cdiv(v, BV),),
        in_specs=[pl.BlockSpec((D_MODEL, BV), lambda i: (0, i))],
        out_specs=pl.BlockSpec((BV, D_PAD), lambda i: (i, 0)),
        out_shape=jax.ShapeDtypeStruct((v, D_PAD), jnp.float32),
    )(tt)


def _gather_body(idx_hbm, table_hbm, out_hbm, idx_v, buf0, buf1, outb,
                 sem0, sem1, *, b_per_w, n_chunks, n_bblk):
    wid = lax.axis_index("s") * _NC + lax.axis_index("c")
    base = wid * b_per_w
    k0 = wid * n_chunks
    pltpu.sync_copy(idx_hbm.at[pl.ds(base, b_per_w)], idx_v)

    bufs = (buf0, buf1)
    sems = (sem0, sem1)
    rowvecs = [lax.iota(jnp.int32, _L) + (l * _L) for l in range(D_MODEL // _L)]

    def start_gather(g, b):
        pltpu.make_async_copy(
            table_hbm.at[idx_v.at[pl.ds(g * CHUNK, CHUNK)]], bufs[b], sems[b]
        ).start()

    def finish(g, b):
        pltpu.make_async_copy(
            table_hbm.at[idx_v.at[pl.ds(g * CHUNK, CHUNK)]], bufs[b], sems[b]
        ).wait()
        buf = bufs[b]

        def xpose(r2, carry):
            for u in range(2):
                r = r2 * 2 + u
                lanevec = jnp.full((_L,), 0, jnp.int32) + r
                for li, rowvec in enumerate(rowvecs):
                    v = buf[r, pl.ds(li * _L, _L)]
                    plsc.store_scatter(outb, [rowvec, lanevec], v)
            return carry

        lax.fori_loop(0, CHUNK // 2, xpose, 0)

        k = k0 + g
        j = k // n_bblk
        bblk = k % n_bblk
        pltpu.sync_copy(outb, out_hbm.at[j, :, pl.ds(bblk * CHUNK, CHUNK)])
        @pl.when(g + 2 < n_chunks)
        def _():
            start_gather(g + 2, b)

    start_gather(0, 0)
    start_gather(1, 1)

    def body(p, carry):
        g = p * 2
        finish(g, 0)
        finish(g + 1, 1)
        return carry

    lax.fori_loop(0, n_chunks // 2, body, 0)


def kernel(tokens, table):
    n_b, n_j = tokens.shape
    idx = tokens.T.reshape(-1).astype(jnp.int32)
    b_total = idx.shape[0]
    b_per_w = b_total // _NW
    n_chunks = b_per_w // CHUNK
    n_bblk = n_b // CHUNK
    table8 = _prep_table(table)
    mesh = plsc.VectorSubcoreMesh(core_axis_name="c", subcore_axis_name="s")
    out3 = pl.kernel(
        functools.partial(_gather_body, b_per_w=b_per_w, n_chunks=n_chunks,
                          n_bblk=n_bblk),
        out_type=jax.ShapeDtypeStruct((n_j, D_MODEL, n_b), jnp.float32),
        mesh=mesh,
        scratch_types=[
            pltpu.VMEM((b_per_w,), jnp.int32),
            pltpu.VMEM((CHUNK, D_PAD), jnp.float32),
            pltpu.VMEM((CHUNK, D_PAD), jnp.float32),
            pltpu.VMEM((D_MODEL, CHUNK), jnp.float32),
            pltpu.SemaphoreType.DMA,
            pltpu.SemaphoreType.DMA,
        ],
        compiler_params=pltpu.CompilerParams(
            use_tc_tiling_on_sc=False, needs_layout_passes=False
        ),
    )(idx, table8)
    return jnp.transpose(out3, (2, 0, 1))
